# baseline (speedup 1.0000x reference)
import functools

import jax
import jax.numpy as jnp
import numpy as np
from jax import lax
from jax.experimental import pallas as pl
from jax.experimental.pallas import tpu as pltpu
from jax.experimental.pallas import tpu_sc as plsc

_N = 10000
_E = 160000
_H = 128
_HEADS = 8
_HD = _H // _HEADS
_CUTOFF = 5.0

_NB = 400
_EB = 640

_NWORK = 32
_EPW = _E // _NWORK
_GCH = 40
_NGCH = _EPW // _GCH

_NSUB = 16
_EPT = _E // _NSUB
_SCH = 80
_NSCH = _EPT // _SCH
_RPS = 624
_RTAIL = _N - _NSUB * _RPS

_F32 = jnp.float32


def _dot(a, b):
    return lax.dot_general(a, b, (((1,), (0,)), ((), ())),
                           precision=lax.Precision.DEFAULT,
                           preferred_element_type=_F32)


def _silu(x):
    return x * jax.nn.sigmoid(x)


_BF = jnp.bfloat16


_I32 = jnp.int32


def _pack2(a, b):
    au = lax.bitcast_convert_type(a.astype(_BF).astype(_F32), _I32)
    bu = lax.bitcast_convert_type(b.astype(_BF).astype(_F32), _I32)
    return jnp.bitwise_and(au, _I32(-65536)) | lax.shift_right_logical(bu, 16)


def _unpk_hi(w):
    return lax.bitcast_convert_type(jnp.bitwise_and(w, _I32(-65536)), _F32)


def _unpk_lo(w):
    return lax.bitcast_convert_type(lax.shift_left(w, 16), _F32)


def _node_body(x_ref, vx_ref, vy_ref, vz_ref, rd_ref, wq_ref, bq_ref, wk_ref,
               bk_ref, wv_ref, bv_ref, wwt_ref, wws_ref, wtt_ref, wts_ref,
               wvec_ref, dtab_ref, stab_ref, vdot_ref, vec3_ref, cutd_ref):
    rd = rd_ref[...]
    cutd_ref[...] = jnp.where(
        rd < _CUTOFF, 0.5 * (jnp.cos(jnp.pi * rd / _CUTOFF) + 1.0), 0.0)
    x = x_ref[...]
    q = _dot(x, wq_ref[...]) + bq_ref[...]
    k = _dot(x, wk_ref[...]) + bk_ref[...]
    v = _dot(x, wv_ref[...]) + bv_ref[...]
    vcs = (vx_ref[...], vy_ref[...], vz_ref[...])
    A = [_dot(vc, wwt_ref[...]) for vc in vcs]
    B = [_dot(vc, wws_ref[...]) for vc in vcs]
    C = [_dot(vc, wtt_ref[...]) for vc in vcs]
    D = [_dot(vc, wts_ref[...]) for vc in vcs]
    dpairs = [(q, A[0]), (A[1], A[2]), (C[0], C[1]), (C[2], D[0]),
              (D[1], D[2])]
    spairs = [(k, v), (vcs[0], vcs[1]), (vcs[2], B[0]), (B[1], B[2])]
    for j, (a, bb) in enumerate(dpairs):
        dtab_ref[:, j * _H:(j + 1) * _H] = _pack2(a, bb)
    for j, (a, bb) in enumerate(spairs):
        stab_ref[:, j * _H:(j + 1) * _H] = _pack2(a, bb)
    vdot = jnp.zeros_like(x)
    for c in range(3):
        v123 = _dot(vcs[c], wvec_ref[...])
        vdot = vdot + v123[:, :_H] * v123[:, _H:2 * _H]
        vec3_ref[:, c * _H:(c + 1) * _H] = v123[:, 2 * _H:]
    vdot_ref[...] = vdot


def _node_call(x, vx, vy, vz, rd, Wq, bq, Wk, bk, Wv, bv, Wwtrg, Wwsrc, Wttrg,
               Wtsrc, Wvec):
    nblk = _N // _NB
    _RD = _E // _H // nblk
    row = lambda r, cdim: pl.BlockSpec((r, cdim), lambda i: (i, 0))
    cspec = pl.BlockSpec((1, _RD, _H), lambda i: (i, 0, 0))
    full = lambda a: pl.BlockSpec(a.shape, lambda i: (0, 0))
    return pl.pallas_call(
        _node_body,
        grid=(nblk,),
        in_specs=[row(_NB, _H)] * 4 + [cspec] + [
            full(Wq), full(bq), full(Wk), full(bk), full(Wv), full(bv),
            full(Wwtrg), full(Wwsrc), full(Wttrg), full(Wtsrc), full(Wvec)],
        out_specs=[row(_NB, 5 * _H), row(_NB, 4 * _H), row(_NB, _H),
                   row(_NB, 3 * _H), cspec],
        out_shape=[jax.ShapeDtypeStruct((_N, 5 * _H), _I32),
                   jax.ShapeDtypeStruct((_N, 4 * _H), _I32),
                   jax.ShapeDtypeStruct((_N, _H), _F32),
                   jax.ShapeDtypeStruct((_N, 3 * _H), _F32),
                   jax.ShapeDtypeStruct((nblk, _RD, _H), _F32)],
    )(x, vx, vy, vz, rd, Wq, bq, Wk, bk, Wv, bv, Wwtrg, Wwsrc, Wttrg, Wtsrc,
      Wvec)


@functools.partial(
    pl.kernel,
    out_type=[jax.ShapeDtypeStruct((_E, 5 * _H), _I32),
              jax.ShapeDtypeStruct((_E, 4 * _H), _I32)],
    mesh=plsc.VectorSubcoreMesh(core_axis_name="c", subcore_axis_name="s"),
    scratch_types=[pltpu.VMEM((_EPW,), jnp.int32),
                   pltpu.VMEM((_EPW,), jnp.int32),
                   pltpu.VMEM((_GCH, 5 * _H), _I32),
                   pltpu.VMEM((_GCH, 5 * _H), _I32),
                   pltpu.VMEM((_GCH, 4 * _H), _I32),
                   pltpu.VMEM((_GCH, 4 * _H), _I32),
                   pltpu.SemaphoreType.DMA,
                   pltpu.SemaphoreType.DMA,
                   pltpu.SemaphoreType.DMA,
                   pltpu.SemaphoreType.DMA],
)
def _gather_call(didx, sidx, dtab, stab, gd_hbm, gs_hbm, idxd_v, idxs_v, bd_0,
                 bd_1, bs_0, bs_1, sg0, sg1, so0, so1):
    c = lax.axis_index("c")
    s = lax.axis_index("s")
    base = (s * 2 + c) * _EPW
    pltpu.sync_copy(didx.at[pl.ds(base, _EPW)], idxd_v)
    pltpu.sync_copy(sidx.at[pl.ds(base, _EPW)], idxs_v)
    bd = (bd_0, bd_1)
    bs = (bs_0, bs_1)
    gsem = (sg0, sg1)
    osem = (so0, so1)

    def issue_g(i, b):
        di = idxd_v.at[pl.ds(i * _GCH, _GCH)]
        si = idxs_v.at[pl.ds(i * _GCH, _GCH)]
        pltpu.async_copy(dtab.at[di], bd[b], gsem[b])
        pltpu.async_copy(stab.at[si], bs[b], gsem[b])

    def wait_g(b):
        pltpu.make_async_copy(dtab.at[pl.ds(0, _GCH)], bd[b], gsem[b]).wait()
        pltpu.make_async_copy(stab.at[pl.ds(0, _GCH)], bs[b], gsem[b]).wait()

    def issue_o(i, b):
        off = base + i * _GCH
        pltpu.async_copy(bd[b], gd_hbm.at[pl.ds(off, _GCH)], osem[b])
        pltpu.async_copy(bs[b], gs_hbm.at[pl.ds(off, _GCH)], osem[b])

    def wait_o(b):
        pltpu.make_async_copy(bd[b], gd_hbm.at[pl.ds(0, _GCH)], osem[b]).wait()
        pltpu.make_async_copy(bs[b], gs_hbm.at[pl.ds(0, _GCH)], osem[b]).wait()

    issue_g(0, 0)

    def body(i, carry):
        for b in (0, 1):
            @pl.when(lax.rem(i, 2) == b)
            def _(b=b):
                @pl.when(i > 0)
                def _():
                    wait_o(1 - b)

                @pl.when(i + 1 < _NGCH)
                def _():
                    issue_g(i + 1, 1 - b)

                wait_g(b)
                issue_o(i, b)
        return carry

    lax.fori_loop(0, _NGCH, body, 0)
    wait_o((_NGCH - 1) % 2)


def _eprep_body(f_ref, wdk_ref, bdk_ref, wdv_ref, bdv_ref, wf_ref, bf_ref,
                ep1_ref, ep2_ref):
    f = f_ref[...]
    dk = _silu(_dot(f, wdk_ref[...]) + bdk_ref[...])
    dv = _silu(_dot(f, wdv_ref[...]) + bdv_ref[...])
    ff = _silu(_dot(f, wf_ref[...]) + bf_ref[...])
    ep1_ref[...] = _pack2(dk, dv)
    ep2_ref[...] = _pack2(ff[:, :_H], ff[:, _H:])


def _eprep_call(f_ij, Wdk, bdk, Wdv, bdv, Wf, bf):
    eblk = _E // _EB
    row = lambda cdim: pl.BlockSpec((_EB, cdim), lambda i: (i, 0))
    full = lambda a: pl.BlockSpec(a.shape, lambda i: (0, 0))
    return pl.pallas_call(
        _eprep_body,
        grid=(eblk,),
        in_specs=[row(_H), full(Wdk), full(bdk), full(Wdv), full(bdv),
                  full(Wf), full(bf)],
        out_specs=[row(_H), row(_H)],
        out_shape=[jax.ShapeDtypeStruct((_E, _H), _I32),
                   jax.ShapeDtypeStruct((_E, _H), _I32)],
    )(f_ij, Wdk, bdk, Wdv, bdv, Wf, bf)


def _edge_body(gd_ref, gs_ref, ep1_ref, ep2_ref, cut_ref, d0_ref, d1_ref,
               d2_ref, ws_ref, bs_ref, m_ref, ve_ref, vm0_ref, vm1_ref,
               vm2_ref, df_ref):
    gd = gd_ref[...]
    gsw = gs_ref[...]
    wd = lambda j: gd[:, j * _H:(j + 1) * _H]
    wsd = lambda j: gsw[:, j * _H:(j + 1) * _H]
    gq = _unpk_hi(wd(0))
    gA = (_unpk_lo(wd(0)), _unpk_hi(wd(1)), _unpk_lo(wd(1)))
    gC = (_unpk_hi(wd(2)), _unpk_lo(wd(2)), _unpk_hi(wd(3)))
    gD = (_unpk_lo(wd(3)), _unpk_hi(wd(4)), _unpk_lo(wd(4)))
    gk = _unpk_hi(wsd(0))
    gv = _unpk_lo(wsd(0))
    gvec = (_unpk_hi(wsd(1)), _unpk_lo(wsd(1)), _unpk_hi(wsd(2)))
    gB = (_unpk_lo(wsd(2)), _unpk_hi(wsd(3)), _unpk_lo(wsd(3)))
    ep1 = ep1_ref[...]
    ep2 = ep2_ref[...]
    dk = _unpk_hi(ep1)
    dv = _unpk_lo(ep1)
    f1 = _unpk_hi(ep2)
    f2 = _unpk_lo(ep2)
    qkd = gq * gk * dk
    attn = _dot(qkd, m_ref[...])
    attn = _silu(attn) * cut_ref[...]
    v_e = gv * dv * attn
    ve_ref[...] = v_e
    sa = _silu(_dot(v_e, ws_ref[...]) + bs_ref[...])
    s1 = sa[:, :_H]
    s2 = sa[:, _H:]
    ds = (d0_ref[...], d1_ref[...], d2_ref[...])
    vm_refs = (vm0_ref, vm1_ref, vm2_ref)
    ad = jnp.zeros_like(s1)
    bd = jnp.zeros_like(s1)
    cd = jnp.zeros_like(s1)
    dd = jnp.zeros_like(s1)
    for c in range(3):
        vm_refs[c][...] = gvec[c] * s1 + s2 * ds[c]
        ad = ad + gA[c] * ds[c]
        bd = bd + gB[c] * ds[c]
        cd = cd + gC[c] * ds[c]
        dd = dd + gD[c] * ds[c]
    w_dot = jnp.zeros_like(s1)
    t_dot = jnp.zeros_like(s1)
    for c in range(3):
        w_dot = w_dot + (gA[c] - ad * ds[c]) * (gB[c] - bd * ds[c])
        t_dot = t_dot + (gC[c] - cd * ds[c]) * (gD[c] - dd * ds[c])
    df_ref[...] = f1 * w_dot + f2 * t_dot


def _edge_call(gd, gs, ep1, ep2, cut2, d0, d1, d2, Ws, bs, Mmat):
    eblk = _E // _EB
    row = lambda cdim: pl.BlockSpec((_EB, cdim), lambda i: (i, 0))
    full = lambda a: pl.BlockSpec(a.shape, lambda i: (0, 0))
    outs = [jax.ShapeDtypeStruct((_E, _H), _F32)] * 5
    return pl.pallas_call(
        _edge_body,
        grid=(eblk,),
        in_specs=[row(5 * _H), row(4 * _H), row(_H), row(_H), row(1), row(1),
                  row(1), row(1), full(Ws), full(bs), full(Mmat)],
        out_specs=[row(_H)] * 5,
        out_shape=outs,
    )(gd, gs, ep1, ep2, cut2, d0, d1, d2, Ws, bs, Mmat)


def _scatter_round(in_hbm, out_hbm, zeros_hbm, idx_v, r0, r1, sl0, sl1, acc,
                   s):
    pltpu.sync_copy(zeros_hbm.at[pl.ds(s * _RPS, _RPS)],
                    acc.at[pl.ds(s * _RPS, _RPS)])

    @pl.when(s == 0)
    def _():
        pltpu.sync_copy(zeros_hbm.at[pl.ds(_NSUB * _RPS, _RTAIL)],
                        acc.at[pl.ds(_NSUB * _RPS, _RTAIL)])

    plsc.subcore_barrier()
    bufs = (r0, r1)
    sems = (sl0, sl1)

    def issue_l(i, b):
        pltpu.async_copy(in_hbm.at[pl.ds(s * _EPT + i * _SCH, _SCH)], bufs[b],
                         sems[b])

    def wait_l(b):
        pltpu.make_async_copy(in_hbm.at[pl.ds(0, _SCH)], bufs[b],
                              sems[b]).wait()

    issue_l(0, 0)

    def body(i, carry):
        for b in (0, 1):
            @pl.when(lax.rem(i, 2) == b)
            def _(b=b):
                @pl.when(i + 1 < _NSCH)
                def _():
                    issue_l(i + 1, 1 - b)

                wait_l(b)
                pltpu.sync_copy(bufs[b], acc.at[idx_v.at[i]], add=True)
        return carry

    lax.fori_loop(0, _NSCH, body, 0)
    plsc.subcore_barrier()
    pltpu.sync_copy(acc.at[pl.ds(s * _RPS, _RPS)],
                    out_hbm.at[pl.ds(s * _RPS, _RPS)])

    @pl.when(s == 0)
    def _():
        pltpu.sync_copy(acc.at[pl.ds(_NSUB * _RPS, _RTAIL)],
                        out_hbm.at[pl.ds(_NSUB * _RPS, _RTAIL)])

    plsc.subcore_barrier()


@functools.partial(
    pl.kernel,
    out_type=[jax.ShapeDtypeStruct((_N, _H), _F32)] * 4,
    mesh=plsc.VectorSubcoreMesh(core_axis_name="c", subcore_axis_name="s"),
    scratch_types=[pltpu.VMEM((_NSCH, _SCH), jnp.int32),
                   pltpu.VMEM((_SCH, _H), _F32),
                   pltpu.VMEM((_SCH, _H), _F32),
                   pltpu.VMEM_SHARED((_N, _H), _F32),
                   pltpu.SemaphoreType.DMA,
                   pltpu.SemaphoreType.DMA],
)
def _scatter_call(didx3, ve, vm0, vm1, vm2, zeros, xagg, va0, va1, va2, idx_v,
                  r0, r1, acc, sl0, sl1):
    c = lax.axis_index("c")
    s = lax.axis_index("s")
    pltpu.sync_copy(didx3.at[s], idx_v)

    @pl.when(c == 0)
    def _():
        _scatter_round(ve, xagg, zeros, idx_v, r0, r1, sl0, sl1, acc, s)
        _scatter_round(vm0, va0, zeros, idx_v, r0, r1, sl0, sl1, acc, s)

    @pl.when(c == 1)
    def _():
        _scatter_round(vm1, va1, zeros, idx_v, r0, r1, sl0, sl1, acc, s)
        _scatter_round(vm2, va2, zeros, idx_v, r0, r1, sl0, sl1, acc, s)


def _final_body(xa_ref, vdot_ref, vec3_ref, va0_ref, va1_ref, va2_ref, wo_ref,
                bo_ref, dx_ref, dv0_ref, dv1_ref, dv2_ref):
    o = _dot(xa_ref[...], wo_ref[...]) + bo_ref[...]
    o1 = o[:, :_H]
    dx_ref[...] = vdot_ref[...] * o[:, _H:2 * _H] + o[:, 2 * _H:]
    va = (va0_ref, va1_ref, va2_ref)
    dv = (dv0_ref, dv1_ref, dv2_ref)
    for c in range(3):
        dv[c][...] = vec3_ref[:, c * _H:(c + 1) * _H] * o1 + va[c][...]


def _final_call(x_agg, vec_dot, vec3, va0, va1, va2, Wo, bo):
    nblk = _N // _NB
    row = lambda cdim: pl.BlockSpec((_NB, cdim), lambda i: (i, 0))
    full = lambda a: pl.BlockSpec(a.shape, lambda i: (0, 0))
    return pl.pallas_call(
        _final_body,
        grid=(nblk,),
        in_specs=[row(_H), row(_H), row(3 * _H), row(_H), row(_H), row(_H),
                  full(Wo), full(bo)],
        out_specs=[row(_H)] * 4,
        out_shape=[jax.ShapeDtypeStruct((_N, _H), _F32)] * 4,
    )(x_agg, vec_dot, vec3, va0, va1, va2, Wo, bo)


def kernel(x, vec, edge_index, r_ij, f_ij, d_ij, Wvec, Wq, bq, Wk, bk, Wv, bv,
           Wdk, bdk, Wdv, bdv, Ws, bs, Wf, bf, Wwsrc, Wwtrg, Wtsrc, Wttrg, Wo,
           bo):
    src = edge_index[0]
    dst = edge_index[1]
    vx, vy, vz = vec[:, 0, :], vec[:, 1, :], vec[:, 2, :]
    rd = r_ij.reshape(25, _E // _H // 25, _H)
    d0, d1, d2 = d_ij[:, 0:1], d_ij[:, 1:2], d_ij[:, 2:3]
    b = lambda a: a.reshape(1, -1)

    dtab, stab, vec_dot, vec3, cutd = _node_call(
        x, vx, vy, vz, rd, Wq, b(bq), Wk, b(bk), Wv, b(bv), Wwtrg, Wwsrc,
        Wttrg, Wtsrc, Wvec)
    cut2 = cutd.reshape(_E, 1)

    ep1, ep2 = _eprep_call(f_ij, Wdk, b(bdk), Wdv, b(bdv), Wf, b(bf))
    gd, gs = _gather_call(dst, src, dtab, stab)

    mmat = jnp.asarray(np.kron(np.eye(_HEADS, dtype=np.float32),
                               np.ones((_HD, _HD), np.float32)))
    v_e, vm0, vm1, vm2, df_ij = _edge_call(
        gd, gs, ep1, ep2, cut2, d0, d1, d2, Ws, b(bs), mmat)

    zeros = jnp.zeros((_N, _H), _F32)
    dst3 = dst.reshape(_NSUB, _NSCH, _SCH)
    x_agg, va0, va1, va2 = _scatter_call(dst3, v_e, vm0, vm1, vm2, zeros)

    dx, dv0, dv1, dv2 = _final_call(x_agg, vec_dot, vec3, va0, va1, va2, Wo,
                                    b(bo))
    dvec = jnp.stack([dv0, dv1, dv2], axis=1)
    return (dx, dvec, df_ij)


# speedup vs baseline: 1.0651x; 1.0651x over previous
import functools

import jax
import jax.numpy as jnp
import numpy as np
from jax import lax
from jax.experimental import pallas as pl
from jax.experimental.pallas import tpu as pltpu
from jax.experimental.pallas import tpu_sc as plsc

_N = 10000
_E = 160000
_H = 128
_HEADS = 8
_HD = _H // _HEADS
_CUTOFF = 5.0

_NB = 400
_EB = 640

_NWORK = 32
_EPW = _E // _NWORK
_GCH = 40
_NGCH = _EPW // _GCH

_NSUB = 16
_EPT = _E // _NSUB
_SCH = 80
_NSCH = _EPT // _SCH
_RPS = 624
_RTAIL = _N - _NSUB * _RPS

_F32 = jnp.float32


def _dot(a, b):
    return lax.dot_general(a, b, (((1,), (0,)), ((), ())),
                           precision=lax.Precision.DEFAULT,
                           preferred_element_type=_F32)


def _silu(x):
    return x * jax.nn.sigmoid(x)


_BF = jnp.bfloat16


_I32 = jnp.int32


def _pack2(a, b):
    au = lax.bitcast_convert_type(a.astype(_BF).astype(_F32), _I32)
    bu = lax.bitcast_convert_type(b.astype(_BF).astype(_F32), _I32)
    return jnp.bitwise_and(au, _I32(-65536)) | lax.shift_right_logical(bu, 16)


def _unpk_hi(w):
    return lax.bitcast_convert_type(jnp.bitwise_and(w, _I32(-65536)), _F32)


def _unpk_lo(w):
    return lax.bitcast_convert_type(lax.shift_left(w, 16), _F32)


def _node_body(x_ref, vx_ref, vy_ref, vz_ref, rd_ref, wq_ref, bq_ref, wk_ref,
               bk_ref, wv_ref, bv_ref, wwt_ref, wws_ref, wtt_ref, wts_ref,
               wvec_ref, dtab_ref, stab_ref, vdot_ref, vec3_ref, cutd_ref):
    rd = rd_ref[...]
    cutd_ref[...] = jnp.where(
        rd < _CUTOFF, 0.5 * (jnp.cos(jnp.pi * rd / _CUTOFF) + 1.0), 0.0)
    x = x_ref[...]
    q = _dot(x, wq_ref[...]) + bq_ref[...]
    k = _dot(x, wk_ref[...]) + bk_ref[...]
    v = _dot(x, wv_ref[...]) + bv_ref[...]
    vcs = (vx_ref[...], vy_ref[...], vz_ref[...])
    A = [_dot(vc, wwt_ref[...]) for vc in vcs]
    B = [_dot(vc, wws_ref[...]) for vc in vcs]
    C = [_dot(vc, wtt_ref[...]) for vc in vcs]
    D = [_dot(vc, wts_ref[...]) for vc in vcs]
    dpairs = [(q, A[0]), (A[1], A[2]), (C[0], C[1]), (C[2], D[0]),
              (D[1], D[2])]
    spairs = [(k, v), (vcs[0], vcs[1]), (vcs[2], B[0]), (B[1], B[2])]
    for j, (a, bb) in enumerate(dpairs):
        dtab_ref[:, j * _H:(j + 1) * _H] = _pack2(a, bb)
    for j, (a, bb) in enumerate(spairs):
        stab_ref[:, j * _H:(j + 1) * _H] = _pack2(a, bb)
    vdot = jnp.zeros_like(x)
    for c in range(3):
        v123 = _dot(vcs[c], wvec_ref[...])
        vdot = vdot + v123[:, :_H] * v123[:, _H:2 * _H]
        vec3_ref[:, c * _H:(c + 1) * _H] = v123[:, 2 * _H:]
    vdot_ref[...] = vdot


def _node_call(x, vx, vy, vz, rd, Wq, bq, Wk, bk, Wv, bv, Wwtrg, Wwsrc, Wttrg,
               Wtsrc, Wvec):
    nblk = _N // _NB
    _RD = _E // _H // nblk
    row = lambda r, cdim: pl.BlockSpec((r, cdim), lambda i: (i, 0))
    cspec = pl.BlockSpec((1, _RD, _H), lambda i: (i, 0, 0))
    full = lambda a: pl.BlockSpec(a.shape, lambda i: (0, 0))
    return pl.pallas_call(
        _node_body,
        grid=(nblk,),
        in_specs=[row(_NB, _H)] * 4 + [cspec] + [
            full(Wq), full(bq), full(Wk), full(bk), full(Wv), full(bv),
            full(Wwtrg), full(Wwsrc), full(Wttrg), full(Wtsrc), full(Wvec)],
        out_specs=[row(_NB, 5 * _H), row(_NB, 4 * _H), row(_NB, _H),
                   row(_NB, 3 * _H), cspec],
        out_shape=[jax.ShapeDtypeStruct((_N, 5 * _H), _I32),
                   jax.ShapeDtypeStruct((_N, 4 * _H), _I32),
                   jax.ShapeDtypeStruct((_N, _H), _F32),
                   jax.ShapeDtypeStruct((_N, 3 * _H), _F32),
                   jax.ShapeDtypeStruct((nblk, _RD, _H), _F32)],
    )(x, vx, vy, vz, rd, Wq, bq, Wk, bk, Wv, bv, Wwtrg, Wwsrc, Wttrg, Wtsrc,
      Wvec)


@functools.partial(
    pl.kernel,
    out_type=[jax.ShapeDtypeStruct((_E, 5 * _H), _I32),
              jax.ShapeDtypeStruct((_E, 4 * _H), _I32)],
    mesh=plsc.VectorSubcoreMesh(core_axis_name="c", subcore_axis_name="s"),
    scratch_types=[pltpu.VMEM((_EPW,), jnp.int32),
                   pltpu.VMEM((_EPW,), jnp.int32),
                   pltpu.VMEM((_GCH, 5 * _H), _I32),
                   pltpu.VMEM((_GCH, 5 * _H), _I32),
                   pltpu.VMEM((_GCH, 4 * _H), _I32),
                   pltpu.VMEM((_GCH, 4 * _H), _I32),
                   pltpu.SemaphoreType.DMA,
                   pltpu.SemaphoreType.DMA,
                   pltpu.SemaphoreType.DMA,
                   pltpu.SemaphoreType.DMA],
)
def _gather_call(didx, sidx, dtab, stab, gd_hbm, gs_hbm, idxd_v, idxs_v, bd_0,
                 bd_1, bs_0, bs_1, sg0, sg1, so0, so1):
    c = lax.axis_index("c")
    s = lax.axis_index("s")
    base = (s * 2 + c) * _EPW
    pltpu.sync_copy(didx.at[pl.ds(base, _EPW)], idxd_v)
    pltpu.sync_copy(sidx.at[pl.ds(base, _EPW)], idxs_v)
    bd = (bd_0, bd_1)
    bs = (bs_0, bs_1)
    gsem = (sg0, sg1)
    osem = (so0, so1)

    def issue_g(i, b):
        di = idxd_v.at[pl.ds(i * _GCH, _GCH)]
        si = idxs_v.at[pl.ds(i * _GCH, _GCH)]
        pltpu.async_copy(dtab.at[di], bd[b], gsem[b])
        pltpu.async_copy(stab.at[si], bs[b], gsem[b])

    def wait_g(b):
        pltpu.make_async_copy(dtab.at[pl.ds(0, _GCH)], bd[b], gsem[b]).wait()
        pltpu.make_async_copy(stab.at[pl.ds(0, _GCH)], bs[b], gsem[b]).wait()

    def issue_o(i, b):
        off = base + i * _GCH
        pltpu.async_copy(bd[b], gd_hbm.at[pl.ds(off, _GCH)], osem[b])
        pltpu.async_copy(bs[b], gs_hbm.at[pl.ds(off, _GCH)], osem[b])

    def wait_o(b):
        pltpu.make_async_copy(bd[b], gd_hbm.at[pl.ds(0, _GCH)], osem[b]).wait()
        pltpu.make_async_copy(bs[b], gs_hbm.at[pl.ds(0, _GCH)], osem[b]).wait()

    issue_g(0, 0)

    def body(i, carry):
        for b in (0, 1):
            @pl.when(lax.rem(i, 2) == b)
            def _(b=b):
                @pl.when(i > 0)
                def _():
                    wait_o(1 - b)

                @pl.when(i + 1 < _NGCH)
                def _():
                    issue_g(i + 1, 1 - b)

                wait_g(b)
                issue_o(i, b)
        return carry

    lax.fori_loop(0, _NGCH, body, 0)
    wait_o((_NGCH - 1) % 2)


def _edge_body(gd_ref, gs_ref, f_ref, cut_ref, d0_ref, d1_ref, d2_ref,
               wdk_ref, bdk_ref, wdv_ref, bdv_ref, wf_ref, bf_ref, ws_ref,
               bs_ref, m_ref, ve_ref, vm0_ref, vm1_ref, vm2_ref, df_ref):
    gd = gd_ref[...]
    gsw = gs_ref[...]
    wd = lambda j: gd[:, j * _H:(j + 1) * _H]
    wsd = lambda j: gsw[:, j * _H:(j + 1) * _H]
    gq = _unpk_hi(wd(0))
    gA = (_unpk_lo(wd(0)), _unpk_hi(wd(1)), _unpk_lo(wd(1)))
    gC = (_unpk_hi(wd(2)), _unpk_lo(wd(2)), _unpk_hi(wd(3)))
    gD = (_unpk_lo(wd(3)), _unpk_hi(wd(4)), _unpk_lo(wd(4)))
    gk = _unpk_hi(wsd(0))
    gv = _unpk_lo(wsd(0))
    gvec = (_unpk_hi(wsd(1)), _unpk_lo(wsd(1)), _unpk_hi(wsd(2)))
    gB = (_unpk_lo(wsd(2)), _unpk_hi(wsd(3)), _unpk_lo(wsd(3)))
    f = f_ref[...]
    dk = _silu(_dot(f, wdk_ref[...]) + bdk_ref[...])
    dv = _silu(_dot(f, wdv_ref[...]) + bdv_ref[...])
    ff = _silu(_dot(f, wf_ref[...]) + bf_ref[...])
    qkd = gq * gk * dk
    attn = _dot(qkd, m_ref[...])
    attn = _silu(attn) * cut_ref[...]
    v_e = gv * dv * attn
    ve_ref[...] = v_e
    sa = _silu(_dot(v_e, ws_ref[...]) + bs_ref[...])
    s1 = sa[:, :_H]
    s2 = sa[:, _H:]
    ds = (d0_ref[...], d1_ref[...], d2_ref[...])
    vm_refs = (vm0_ref, vm1_ref, vm2_ref)
    ad = jnp.zeros_like(s1)
    bd = jnp.zeros_like(s1)
    cd = jnp.zeros_like(s1)
    dd = jnp.zeros_like(s1)
    for c in range(3):
        vm_refs[c][...] = gvec[c] * s1 + s2 * ds[c]
        ad = ad + gA[c] * ds[c]
        bd = bd + gB[c] * ds[c]
        cd = cd + gC[c] * ds[c]
        dd = dd + gD[c] * ds[c]
    w_dot = -ad * bd
    t_dot = -cd * dd
    for c in range(3):
        w_dot = w_dot + gA[c] * gB[c]
        t_dot = t_dot + gC[c] * gD[c]
    df_ref[...] = ff[:, :_H] * w_dot + ff[:, _H:] * t_dot


def _edge_call(gd, gs, f_ij, cut2, d0, d1, d2, Wdk, bdk, Wdv, bdv, Wf, bf,
               Ws, bs, Mmat):
    eblk = _E // _EB
    row = lambda cdim: pl.BlockSpec((_EB, cdim), lambda i: (i, 0))
    full = lambda a: pl.BlockSpec(a.shape, lambda i: (0, 0))
    outs = [jax.ShapeDtypeStruct((_E, _H), _F32)] * 5
    return pl.pallas_call(
        _edge_body,
        grid=(eblk,),
        in_specs=[row(5 * _H), row(4 * _H), row(_H), row(1), row(1), row(1),
                  row(1), full(Wdk), full(bdk), full(Wdv), full(bdv), full(Wf),
                  full(bf), full(Ws), full(bs), full(Mmat)],
        out_specs=[row(_H)] * 5,
        out_shape=outs,
    )(gd, gs, f_ij, cut2, d0, d1, d2, Wdk, bdk, Wdv, bdv, Wf, bf, Ws, bs,
      Mmat)


def _scatter_round(in_hbm, out_hbm, zeros_hbm, idx_v, r0, r1, sl0, sl1, acc,
                   s):
    pltpu.sync_copy(zeros_hbm.at[pl.ds(s * _RPS, _RPS)],
                    acc.at[pl.ds(s * _RPS, _RPS)])

    @pl.when(s == 0)
    def _():
        pltpu.sync_copy(zeros_hbm.at[pl.ds(_NSUB * _RPS, _RTAIL)],
                        acc.at[pl.ds(_NSUB * _RPS, _RTAIL)])

    plsc.subcore_barrier()
    bufs = (r0, r1)
    sems = (sl0, sl1)

    def issue_l(i, b):
        pltpu.async_copy(in_hbm.at[pl.ds(s * _EPT + i * _SCH, _SCH)], bufs[b],
                         sems[b])

    def wait_l(b):
        pltpu.make_async_copy(in_hbm.at[pl.ds(0, _SCH)], bufs[b],
                              sems[b]).wait()

    issue_l(0, 0)

    def body(i, carry):
        for b in (0, 1):
            @pl.when(lax.rem(i, 2) == b)
            def _(b=b):
                @pl.when(i + 1 < _NSCH)
                def _():
                    issue_l(i + 1, 1 - b)

                wait_l(b)
                pltpu.sync_copy(bufs[b], acc.at[idx_v.at[i]], add=True)
        return carry

    lax.fori_loop(0, _NSCH, body, 0)
    plsc.subcore_barrier()
    pltpu.sync_copy(acc.at[pl.ds(s * _RPS, _RPS)],
                    out_hbm.at[pl.ds(s * _RPS, _RPS)])

    @pl.when(s == 0)
    def _():
        pltpu.sync_copy(acc.at[pl.ds(_NSUB * _RPS, _RTAIL)],
                        out_hbm.at[pl.ds(_NSUB * _RPS, _RTAIL)])

    plsc.subcore_barrier()


@functools.partial(
    pl.kernel,
    out_type=[jax.ShapeDtypeStruct((_N, _H), _F32)] * 4,
    mesh=plsc.VectorSubcoreMesh(core_axis_name="c", subcore_axis_name="s"),
    scratch_types=[pltpu.VMEM((_NSCH, _SCH), jnp.int32),
                   pltpu.VMEM((_SCH, _H), _F32),
                   pltpu.VMEM((_SCH, _H), _F32),
                   pltpu.VMEM_SHARED((_N, _H), _F32),
                   pltpu.SemaphoreType.DMA,
                   pltpu.SemaphoreType.DMA],
)
def _scatter_call(didx3, ve, vm0, vm1, vm2, zeros, xagg, va0, va1, va2, idx_v,
                  r0, r1, acc, sl0, sl1):
    c = lax.axis_index("c")
    s = lax.axis_index("s")
    pltpu.sync_copy(didx3.at[s], idx_v)

    @pl.when(c == 0)
    def _():
        _scatter_round(ve, xagg, zeros, idx_v, r0, r1, sl0, sl1, acc, s)
        _scatter_round(vm0, va0, zeros, idx_v, r0, r1, sl0, sl1, acc, s)

    @pl.when(c == 1)
    def _():
        _scatter_round(vm1, va1, zeros, idx_v, r0, r1, sl0, sl1, acc, s)
        _scatter_round(vm2, va2, zeros, idx_v, r0, r1, sl0, sl1, acc, s)


def _final_body(xa_ref, vdot_ref, vec3_ref, va0_ref, va1_ref, va2_ref, wo_ref,
                bo_ref, dx_ref, dv0_ref, dv1_ref, dv2_ref):
    o = _dot(xa_ref[...], wo_ref[...]) + bo_ref[...]
    o1 = o[:, :_H]
    dx_ref[...] = vdot_ref[...] * o[:, _H:2 * _H] + o[:, 2 * _H:]
    va = (va0_ref, va1_ref, va2_ref)
    dv = (dv0_ref, dv1_ref, dv2_ref)
    for c in range(3):
        dv[c][...] = vec3_ref[:, c * _H:(c + 1) * _H] * o1 + va[c][...]


def _final_call(x_agg, vec_dot, vec3, va0, va1, va2, Wo, bo):
    nblk = _N // _NB
    row = lambda cdim: pl.BlockSpec((_NB, cdim), lambda i: (i, 0))
    full = lambda a: pl.BlockSpec(a.shape, lambda i: (0, 0))
    return pl.pallas_call(
        _final_body,
        grid=(nblk,),
        in_specs=[row(_H), row(_H), row(3 * _H), row(_H), row(_H), row(_H),
                  full(Wo), full(bo)],
        out_specs=[row(_H)] * 4,
        out_shape=[jax.ShapeDtypeStruct((_N, _H), _F32)] * 4,
    )(x_agg, vec_dot, vec3, va0, va1, va2, Wo, bo)


def kernel(x, vec, edge_index, r_ij, f_ij, d_ij, Wvec, Wq, bq, Wk, bk, Wv, bv,
           Wdk, bdk, Wdv, bdv, Ws, bs, Wf, bf, Wwsrc, Wwtrg, Wtsrc, Wttrg, Wo,
           bo):
    src = edge_index[0]
    dst = edge_index[1]
    vx, vy, vz = vec[:, 0, :], vec[:, 1, :], vec[:, 2, :]
    rd = r_ij.reshape(25, _E // _H // 25, _H)
    d0, d1, d2 = d_ij[:, 0:1], d_ij[:, 1:2], d_ij[:, 2:3]
    b = lambda a: a.reshape(1, -1)

    dtab, stab, vec_dot, vec3, cutd = _node_call(
        x, vx, vy, vz, rd, Wq, b(bq), Wk, b(bk), Wv, b(bv), Wwtrg, Wwsrc,
        Wttrg, Wtsrc, Wvec)
    cut2 = cutd.reshape(_E, 1)

    gd, gs = _gather_call(dst, src, dtab, stab)

    mmat = jnp.asarray(np.kron(np.eye(_HEADS, dtype=np.float32),
                               np.ones((_HD, _HD), np.float32)))
    v_e, vm0, vm1, vm2, df_ij = _edge_call(
        gd, gs, f_ij, cut2, d0, d1, d2, Wdk, b(bdk), Wdv, b(bdv), Wf, b(bf),
        Ws, b(bs), mmat)

    zeros = jnp.zeros((_N, _H), _F32)
    dst3 = dst.reshape(_NSUB, _NSCH, _SCH)
    x_agg, va0, va1, va2 = _scatter_call(dst3, v_e, vm0, vm1, vm2, zeros)

    dx, dv0, dv1, dv2 = _final_call(x_agg, vec_dot, vec3, va0, va1, va2, Wo,
                                    b(bo))
    dvec = jnp.stack([dv0, dv1, dv2], axis=1)
    return (dx, dvec, df_ij)
